# baseline (speedup 1.0000x reference)
import functools

import jax
import jax.numpy as jnp
from jax import lax
from jax.experimental import pallas as pl
from jax.experimental.pallas import tpu as pltpu
from jax.experimental.pallas import tpu_sc as plsc

N_HASHES = 8
BUCKET = 64
NB = 32
TBLK = 256

_HIGH = lax.Precision.HIGHEST


def _argmax_first(vals, n):
    m = jnp.max(vals, axis=-1, keepdims=True)
    ids = lax.broadcasted_iota(jnp.int32, vals.shape, vals.ndim - 1)
    cand = jnp.where(vals >= m, ids, n)
    return jnp.min(cand, axis=-1)


def _stage_a(x_ref, wqk_ref, bqk_ref, wv_ref, bv_ref, rot_ref,
             qk_ref, rn_ref, v_ref, bkt_ref):
    xb = x_ref[...].astype(jnp.bfloat16)
    qk = lax.dot_general(xb, wqk_ref[...].astype(jnp.bfloat16),
                         (((1,), (1,)), ((), ())),
                         preferred_element_type=jnp.float32) + bqk_ref[...]
    vv = lax.dot_general(xb, wv_ref[...].astype(jnp.bfloat16),
                         (((1,), (1,)), ((), ())),
                         preferred_element_type=jnp.float32) + bv_ref[...]
    qk_ref[...] = qk
    v_ref[...] = vv
    nrm = jnp.sqrt(jnp.sum(qk * qk, axis=1, keepdims=True))
    nrm = jnp.maximum(nrm, 1e-12)
    rn_ref[...] = (1.0 / 32.0) / nrm
    rv = jnp.dot(qk.astype(jnp.bfloat16),
                 rot_ref[...].astype(jnp.bfloat16),
                 preferred_element_type=jnp.float32)
    cols = []
    for h in range(N_HASHES):
        sl = rv[:, h * (NB // 2):(h + 1) * (NB // 2)]
        vals = jnp.concatenate([sl, -sl], axis=1)
        cols.append(_argmax_first(vals, NB).reshape(-1, 1))
    colsmat = jnp.concatenate(cols, axis=1).astype(jnp.float32)
    rr = lax.broadcasted_iota(jnp.int32, (TBLK, TBLK), 0)
    cc = lax.broadcasted_iota(jnp.int32, (TBLK, TBLK), 1)
    eye = (rr == cc).astype(jnp.float32)
    bktT = lax.dot_general(colsmat, eye, (((0,), (0,)), ((), ())),
                           preferred_element_type=jnp.float32)
    bkt_ref[...] = bktT.astype(jnp.int32)


def _sc_sort(bkt_hbm, gch_hbm, buf_ref, rank_ref, out_ref, cnt_ref, off_ref):
    wid = lax.axis_index("s") * 2 + lax.axis_index("c")
    b = wid // N_HASHES
    h = wid % N_HASHES
    lid = lax.broadcasted_iota(jnp.int32, (16,), 0)

    @pl.when(wid < 2 * N_HASHES)
    def _():
        pltpu.sync_copy(bkt_hbm.at[b, h], buf_ref)
        cnt_ref[pl.ds(0, 16)] = jnp.zeros((16,), jnp.int32)
        cnt_ref[pl.ds(16, 16)] = jnp.zeros((16,), jnp.int32)

        def step1(t0, carry):
            base = t0 * 16
            bv = buf_ref[pl.ds(base, 16)]
            tot = jnp.zeros((16,), jnp.int32)
            pre = jnp.zeros((16,), jnp.int32)
            for j in range(16):
                bj = plsc.load_gather(buf_ref, [lid * 0 + (base + j)])
                eq = (bv == bj).astype(jnp.int32)
                tot = tot + eq
                pre = pre + jnp.where(lid > j, eq, 0)
            old = plsc.load_gather(cnt_ref, [bv])
            rank_ref[pl.ds(base, 16)] = old + pre
            plsc.store_scatter(cnt_ref, [bv], old + tot,
                               mask=pre == tot - 1)
            return carry

        lax.fori_loop(0, buf_ref.shape[0] // 16, step1, 0)
        c0 = cnt_ref[pl.ds(0, 16)]
        c1 = cnt_ref[pl.ds(16, 16)]
        off_ref[pl.ds(0, 16)] = plsc.cumsum(c0) - c0
        off_ref[pl.ds(16, 16)] = plsc.cumsum(c1) - c1 + jnp.sum(c0)

        def step2(t0, carry):
            base = t0 * 16
            bv = buf_ref[pl.ds(base, 16)]
            pos = plsc.load_gather(off_ref, [bv]) + rank_ref[pl.ds(base, 16)]
            out_ref[pl.ds(base, 16)] = (pos >> 6) + h * NB
            return carry

        lax.fori_loop(0, buf_ref.shape[0] // 16, step2, 0)
        pltpu.sync_copy(out_ref, gch_hbm.at[b, h])


def _onehot_chunks_t(g):
    kcol = lax.broadcasted_iota(jnp.int32, (N_HASHES * NB, 1), 0)
    u = jnp.zeros((N_HASHES * NB, g.shape[1]), jnp.float32)
    for h in range(N_HASHES):
        u = u + (g[h:h + 1, :] == kcol).astype(jnp.float32)
    return u.astype(jnp.bfloat16)


def _stage_e(qki_ref, qkf_ref, rn_ref, v_ref, gi_ref, gf_ref,
             wout_ref, bout_ref, out_ref):
    nj = qkf_ref.shape[0] // TBLK
    u_i = _onehot_chunks_t(gi_ref[...])
    qki_bf = qki_ref[...].astype(jnp.bfloat16)
    acc = jnp.zeros((TBLK, qkf_ref.shape[1]), jnp.float32)
    for j in range(nj):
        sl = slice(j * TBLK, (j + 1) * TBLK)
        u_j = _onehot_chunks_t(gf_ref[:, sl])
        cb = lax.dot_general(u_i, u_j, (((0,), (0,)), ((), ())),
                             preferred_element_type=jnp.float32)
        qkn_j = (qkf_ref[sl, :] * rn_ref[sl, :]).astype(jnp.bfloat16)
        sb = lax.dot_general(qki_bf, qkn_j, (((1,), (1,)), ((), ())),
                             preferred_element_type=jnp.float32)
        w = (cb * sb).astype(jnp.bfloat16)
        acc = acc + jnp.dot(w, v_ref[sl, :].astype(jnp.bfloat16),
                            preferred_element_type=jnp.float32)
    out_ref[...] = lax.dot_general(
        acc.astype(jnp.bfloat16), wout_ref[...].astype(jnp.bfloat16),
        (((1,), (1,)), ((), ())),
        preferred_element_type=jnp.float32) + bout_ref[...]


def kernel(x, Wqk, bqk, Wv, bv, Wout, bout):
    B, S, D = x.shape
    nblk = S // TBLK
    rot = jax.random.normal(jax.random.key(42), (1, D, N_HASHES, NB // 2),
                            dtype=x.dtype)
    rot2 = rot.reshape(D, N_HASHES * (NB // 2))

    qk, rn, v, bkt = pl.pallas_call(
        _stage_a,
        grid=(B, nblk),
        in_specs=[
            pl.BlockSpec((None, TBLK, D), lambda b, i: (b, i, 0)),
            pl.BlockSpec((D, D), lambda b, i: (0, 0)),
            pl.BlockSpec((1, D), lambda b, i: (0, 0)),
            pl.BlockSpec((D, D), lambda b, i: (0, 0)),
            pl.BlockSpec((1, D), lambda b, i: (0, 0)),
            pl.BlockSpec((D, N_HASHES * (NB // 2)), lambda b, i: (0, 0)),
        ],
        out_specs=[
            pl.BlockSpec((None, TBLK, D), lambda b, i: (b, i, 0)),
            pl.BlockSpec((None, TBLK, 1), lambda b, i: (b, i, 0)),
            pl.BlockSpec((None, TBLK, D), lambda b, i: (b, i, 0)),
            pl.BlockSpec((None, N_HASHES, TBLK), lambda b, i: (b, 0, i)),
        ],
        out_shape=[
            jax.ShapeDtypeStruct((B, S, D), jnp.float32),
            jax.ShapeDtypeStruct((B, S, 1), jnp.float32),
            jax.ShapeDtypeStruct((B, S, D), jnp.float32),
            jax.ShapeDtypeStruct((B, N_HASHES, S), jnp.int32),
        ],
    )(x, Wqk, bqk.reshape(1, D), Wv, bv.reshape(1, D), rot2)

    gch = pl.kernel(
        _sc_sort,
        out_type=jax.ShapeDtypeStruct((B, N_HASHES, S), jnp.int32),
        mesh=plsc.VectorSubcoreMesh(core_axis_name="c", subcore_axis_name="s"),
        scratch_types=[
            pltpu.VMEM((S,), jnp.int32),
            pltpu.VMEM((S,), jnp.int32),
            pltpu.VMEM((S,), jnp.int32),
            pltpu.VMEM((NB,), jnp.int32),
            pltpu.VMEM((NB,), jnp.int32),
        ],
        compiler_params=pltpu.CompilerParams(needs_layout_passes=False),
    )(bkt)

    out = pl.pallas_call(
        _stage_e,
        grid=(B, nblk),
        in_specs=[
            pl.BlockSpec((None, TBLK, D), lambda b, i: (b, i, 0)),
            pl.BlockSpec((None, S, D), lambda b, i: (b, 0, 0)),
            pl.BlockSpec((None, S, 1), lambda b, i: (b, 0, 0)),
            pl.BlockSpec((None, S, D), lambda b, i: (b, 0, 0)),
            pl.BlockSpec((None, N_HASHES, TBLK), lambda b, i: (b, 0, i)),
            pl.BlockSpec((None, N_HASHES, S), lambda b, i: (b, 0, 0)),
            pl.BlockSpec((D, D), lambda b, i: (0, 0)),
            pl.BlockSpec((1, D), lambda b, i: (0, 0)),
        ],
        out_specs=pl.BlockSpec((None, TBLK, D), lambda b, i: (b, i, 0)),
        out_shape=jax.ShapeDtypeStruct((B, S, D), jnp.float32),
        compiler_params=pltpu.CompilerParams(
            dimension_semantics=("parallel", "arbitrary")),
    )(qk, qk, rn, v, gch, gch, Wout, bout.reshape(1, D))

    return out


# speedup vs baseline: 1.0215x; 1.0215x over previous
import functools

import jax
import jax.numpy as jnp
from jax import lax
from jax.experimental import pallas as pl
from jax.experimental.pallas import tpu as pltpu
from jax.experimental.pallas import tpu_sc as plsc

N_HASHES = 8
BUCKET = 64
NB = 32
TBLK = 256

_HIGH = lax.Precision.HIGHEST


def _argmax_first(vals, n):
    m = jnp.max(vals, axis=-1, keepdims=True)
    ids = lax.broadcasted_iota(jnp.int32, vals.shape, vals.ndim - 1)
    cand = jnp.where(vals >= m, ids, n)
    return jnp.min(cand, axis=-1)


def _stage_a(x_ref, wqk_ref, bqk_ref, wv_ref, bv_ref, rot_ref,
             qk_ref, qkn_ref, v_ref, bkt_ref):
    xb = x_ref[...]
    qk = lax.dot_general(xb, wqk_ref[...], (((1,), (1,)), ((), ())),
                         preferred_element_type=jnp.float32) + bqk_ref[...]
    vv = lax.dot_general(xb, wv_ref[...], (((1,), (1,)), ((), ())),
                         preferred_element_type=jnp.float32) + bv_ref[...]
    qkbf = qk.astype(jnp.bfloat16)
    qk_ref[...] = qkbf
    v_ref[...] = vv.astype(jnp.bfloat16)
    nrm = jnp.sqrt(jnp.sum(qk * qk, axis=1, keepdims=True))
    nrm = jnp.maximum(nrm, 1e-12)
    qkn_ref[...] = (qk * ((1.0 / 32.0) / nrm)).astype(jnp.bfloat16)
    rv = jnp.dot(qkbf, rot_ref[...], preferred_element_type=jnp.float32)
    cols = []
    for h in range(N_HASHES):
        sl = rv[:, h * (NB // 2):(h + 1) * (NB // 2)]
        vals = jnp.concatenate([sl, -sl], axis=1)
        cols.append(_argmax_first(vals, NB).reshape(-1, 1))
    colsmat = jnp.concatenate(cols, axis=1).astype(jnp.float32)
    rr = lax.broadcasted_iota(jnp.int32, (TBLK, TBLK), 0)
    cc = lax.broadcasted_iota(jnp.int32, (TBLK, TBLK), 1)
    eye = (rr == cc).astype(jnp.float32)
    bktT = lax.dot_general(colsmat, eye, (((0,), (0,)), ((), ())),
                           preferred_element_type=jnp.float32)
    bkt_ref[...] = bktT.astype(jnp.int32)


def _sc_sort(bkt_hbm, gch_hbm, buf_ref, rank_ref, out_ref, cnt_ref, off_ref):
    wid = lax.axis_index("s") * 2 + lax.axis_index("c")
    b = wid // N_HASHES
    h = wid % N_HASHES
    lid = lax.broadcasted_iota(jnp.int32, (16,), 0)

    @pl.when(wid < 2 * N_HASHES)
    def _():
        pltpu.sync_copy(bkt_hbm.at[b, h], buf_ref)
        cnt_ref[pl.ds(0, 16)] = jnp.zeros((16,), jnp.int32)
        cnt_ref[pl.ds(16, 16)] = jnp.zeros((16,), jnp.int32)

        def step1(t0, carry):
            base = t0 * 16
            bv = buf_ref[pl.ds(base, 16)]
            tot = jnp.zeros((16,), jnp.int32)
            pre = jnp.zeros((16,), jnp.int32)
            for j in range(16):
                bj = plsc.load_gather(buf_ref, [lid * 0 + (base + j)])
                eq = (bv == bj).astype(jnp.int32)
                tot = tot + eq
                pre = pre + jnp.where(lid > j, eq, 0)
            old = plsc.load_gather(cnt_ref, [bv])
            rank_ref[pl.ds(base, 16)] = old + pre
            plsc.store_scatter(cnt_ref, [bv], old + tot,
                               mask=pre == tot - 1)
            return carry

        lax.fori_loop(0, buf_ref.shape[0] // 16, step1, 0)
        c0 = cnt_ref[pl.ds(0, 16)]
        c1 = cnt_ref[pl.ds(16, 16)]
        off_ref[pl.ds(0, 16)] = plsc.cumsum(c0) - c0
        off_ref[pl.ds(16, 16)] = plsc.cumsum(c1) - c1 + jnp.sum(c0)

        def step2(t0, carry):
            base = t0 * 16
            bv = buf_ref[pl.ds(base, 16)]
            pos = plsc.load_gather(off_ref, [bv]) + rank_ref[pl.ds(base, 16)]
            out_ref[pl.ds(base, 16)] = (pos >> 6) + h * NB
            return carry

        lax.fori_loop(0, buf_ref.shape[0] // 16, step2, 0)
        pltpu.sync_copy(out_ref, gch_hbm.at[b, h])


def _onehot_chunks_t(g):
    kcol = lax.broadcasted_iota(jnp.int32, (N_HASHES * NB, 1), 0)
    u = jnp.zeros((N_HASHES * NB, g.shape[1]), jnp.float32)
    for h in range(N_HASHES):
        u = u + (g[h:h + 1, :] == kcol).astype(jnp.float32)
    return u.astype(jnp.bfloat16)


def _stage_e(qki_ref, qkn_ref, v_ref, gi_ref, gf_ref,
             wout_ref, bout_ref, out_ref, ut_ref):
    nj = qkn_ref.shape[0] // TBLK

    @pl.when(pl.program_id(1) == 0)
    def _():
        for j in range(nj):
            sl = slice(j * TBLK, (j + 1) * TBLK)
            ut_ref[:, sl] = _onehot_chunks_t(gf_ref[:, sl])

    u_i = _onehot_chunks_t(gi_ref[...])
    qki_bf = qki_ref[...]
    acc = jnp.zeros((TBLK, qkn_ref.shape[1]), jnp.float32)
    for j in range(nj):
        sl = slice(j * TBLK, (j + 1) * TBLK)
        u_j = ut_ref[:, sl]
        cb = lax.dot_general(u_i, u_j, (((0,), (0,)), ((), ())),
                             preferred_element_type=jnp.float32)
        sb = lax.dot_general(qki_bf, qkn_ref[sl, :], (((1,), (1,)), ((), ())),
                             preferred_element_type=jnp.float32)
        w = (cb * sb).astype(jnp.bfloat16)
        acc = acc + jnp.dot(w, v_ref[sl, :],
                            preferred_element_type=jnp.float32)
    out_ref[...] = lax.dot_general(
        acc.astype(jnp.bfloat16), wout_ref[...], (((1,), (1,)), ((), ())),
        preferred_element_type=jnp.float32) + bout_ref[...]


def kernel(x, Wqk, bqk, Wv, bv, Wout, bout):
    B, S, D = x.shape
    nblk = S // TBLK
    rot = jax.random.normal(jax.random.key(42), (1, D, N_HASHES, NB // 2),
                            dtype=x.dtype)
    rot2 = rot.reshape(D, N_HASHES * (NB // 2))
    xbf = x.astype(jnp.bfloat16)
    wqkbf = Wqk.astype(jnp.bfloat16)
    wvbf = Wv.astype(jnp.bfloat16)
    woutbf = Wout.astype(jnp.bfloat16)
    rotbf = rot2.astype(jnp.bfloat16)

    qk, qkn, v, bkt = pl.pallas_call(
        _stage_a,
        grid=(B, nblk),
        in_specs=[
            pl.BlockSpec((None, TBLK, D), lambda b, i: (b, i, 0)),
            pl.BlockSpec((D, D), lambda b, i: (0, 0)),
            pl.BlockSpec((1, D), lambda b, i: (0, 0)),
            pl.BlockSpec((D, D), lambda b, i: (0, 0)),
            pl.BlockSpec((1, D), lambda b, i: (0, 0)),
            pl.BlockSpec((D, N_HASHES * (NB // 2)), lambda b, i: (0, 0)),
        ],
        out_specs=[
            pl.BlockSpec((None, TBLK, D), lambda b, i: (b, i, 0)),
            pl.BlockSpec((None, TBLK, D), lambda b, i: (b, i, 0)),
            pl.BlockSpec((None, TBLK, D), lambda b, i: (b, i, 0)),
            pl.BlockSpec((None, N_HASHES, TBLK), lambda b, i: (b, 0, i)),
        ],
        out_shape=[
            jax.ShapeDtypeStruct((B, S, D), jnp.bfloat16),
            jax.ShapeDtypeStruct((B, S, D), jnp.bfloat16),
            jax.ShapeDtypeStruct((B, S, D), jnp.bfloat16),
            jax.ShapeDtypeStruct((B, N_HASHES, S), jnp.int32),
        ],
        compiler_params=pltpu.CompilerParams(
            dimension_semantics=("parallel", "parallel")),
    )(xbf, wqkbf, bqk.reshape(1, D), wvbf, bv.reshape(1, D), rotbf)

    gch = pl.kernel(
        _sc_sort,
        out_type=jax.ShapeDtypeStruct((B, N_HASHES, S), jnp.int32),
        mesh=plsc.VectorSubcoreMesh(core_axis_name="c", subcore_axis_name="s"),
        scratch_types=[
            pltpu.VMEM((S,), jnp.int32),
            pltpu.VMEM((S,), jnp.int32),
            pltpu.VMEM((S,), jnp.int32),
            pltpu.VMEM((NB,), jnp.int32),
            pltpu.VMEM((NB,), jnp.int32),
        ],
        compiler_params=pltpu.CompilerParams(needs_layout_passes=False),
    )(bkt)

    out = pl.pallas_call(
        _stage_e,
        grid=(B, nblk),
        in_specs=[
            pl.BlockSpec((None, TBLK, D), lambda b, i: (b, i, 0)),
            pl.BlockSpec((None, S, D), lambda b, i: (b, 0, 0)),
            pl.BlockSpec((None, S, D), lambda b, i: (b, 0, 0)),
            pl.BlockSpec((None, N_HASHES, TBLK), lambda b, i: (b, 0, i)),
            pl.BlockSpec((None, N_HASHES, S), lambda b, i: (b, 0, 0)),
            pl.BlockSpec((D, D), lambda b, i: (0, 0)),
            pl.BlockSpec((1, D), lambda b, i: (0, 0)),
        ],
        out_specs=pl.BlockSpec((None, TBLK, D), lambda b, i: (b, i, 0)),
        out_shape=jax.ShapeDtypeStruct((B, S, D), jnp.float32),
        scratch_shapes=[pltpu.VMEM((N_HASHES * NB, S), jnp.bfloat16)],
        compiler_params=pltpu.CompilerParams(
            dimension_semantics=("parallel", "arbitrary")),
    )(qk, qkn, v, gch, gch, woutbf, bout.reshape(1, D))

    return out


# speedup vs baseline: 1.1721x; 1.1474x over previous
import functools

import jax
import jax.numpy as jnp
from jax import lax
from jax.experimental import pallas as pl
from jax.experimental.pallas import tpu as pltpu
from jax.experimental.pallas import tpu_sc as plsc

N_HASHES = 8
BUCKET = 64
NB = 32
TBLK = 256

_HIGH = lax.Precision.HIGHEST


def _argmax_first(vals, n):
    m = jnp.max(vals, axis=-1, keepdims=True)
    ids = lax.broadcasted_iota(jnp.int32, vals.shape, vals.ndim - 1)
    cand = jnp.where(vals >= m, ids, n)
    return jnp.min(cand, axis=-1)


def _stage_a(x_ref, wqk_ref, bqk_ref, wv_ref, bv_ref, rot_ref,
             qk_ref, qkn_ref, v_ref, bkt_ref):
    xb = x_ref[...]
    qk = lax.dot_general(xb, wqk_ref[...], (((1,), (1,)), ((), ())),
                         preferred_element_type=jnp.float32) + bqk_ref[...]
    vv = lax.dot_general(xb, wv_ref[...], (((1,), (1,)), ((), ())),
                         preferred_element_type=jnp.float32) + bv_ref[...]
    qkbf = qk.astype(jnp.bfloat16)
    qk_ref[...] = qkbf
    v_ref[...] = vv.astype(jnp.bfloat16)
    nrm = jnp.sqrt(jnp.sum(qk * qk, axis=1, keepdims=True))
    nrm = jnp.maximum(nrm, 1e-12)
    qkn_ref[...] = (qk * ((1.0 / 32.0) / nrm)).astype(jnp.bfloat16)
    rv = jnp.dot(qkbf, rot_ref[...], preferred_element_type=jnp.float32)
    cols = []
    for h in range(N_HASHES):
        sl = rv[:, h * (NB // 2):(h + 1) * (NB // 2)]
        vals = jnp.concatenate([sl, -sl], axis=1)
        cols.append(_argmax_first(vals, NB).reshape(-1, 1))
    colsmat = jnp.concatenate(cols, axis=1).astype(jnp.float32)
    rr = lax.broadcasted_iota(jnp.int32, (TBLK, TBLK), 0)
    cc = lax.broadcasted_iota(jnp.int32, (TBLK, TBLK), 1)
    eye = (rr == cc).astype(jnp.float32)
    bktT = lax.dot_general(colsmat, eye, (((0,), (0,)), ((), ())),
                           preferred_element_type=jnp.float32)
    bkt_ref[...] = bktT.astype(jnp.int32)


def _sc_sort(bkt_hbm, gch_hbm, buf_ref, rank_ref, out_ref, cnt_ref, off_ref):
    wid = lax.axis_index("s") * 2 + lax.axis_index("c")
    b = wid // N_HASHES
    h = wid % N_HASHES
    lid = lax.broadcasted_iota(jnp.int32, (16,), 0)

    @pl.when(wid < 2 * N_HASHES)
    def _():
        pltpu.sync_copy(bkt_hbm.at[b, h], buf_ref)
        cnt_ref[pl.ds(0, 16)] = jnp.zeros((16,), jnp.int32)
        cnt_ref[pl.ds(16, 16)] = jnp.zeros((16,), jnp.int32)

        def step1(t0, carry):
            base = t0 * 16
            bv = buf_ref[pl.ds(base, 16)]
            tot = jnp.zeros((16,), jnp.int32)
            pre = jnp.zeros((16,), jnp.int32)
            for j in range(16):
                bj = plsc.load_gather(buf_ref, [lid * 0 + (base + j)])
                eq = (bv == bj).astype(jnp.int32)
                tot = tot + eq
                pre = pre + jnp.where(lid > j, eq, 0)
            old = plsc.load_gather(cnt_ref, [bv])
            rank_ref[pl.ds(base, 16)] = old + pre
            plsc.store_scatter(cnt_ref, [bv], old + tot,
                               mask=pre == tot - 1)
            return carry

        lax.fori_loop(0, buf_ref.shape[0] // 16, step1, 0)
        c0 = cnt_ref[pl.ds(0, 16)]
        c1 = cnt_ref[pl.ds(16, 16)]
        off_ref[pl.ds(0, 16)] = plsc.cumsum(c0) - c0
        off_ref[pl.ds(16, 16)] = plsc.cumsum(c1) - c1 + jnp.sum(c0)

        def step2(t0, carry):
            base = t0 * 16
            bv = buf_ref[pl.ds(base, 16)]
            pos = plsc.load_gather(off_ref, [bv]) + rank_ref[pl.ds(base, 16)]
            out_ref[pl.ds(base, 16)] = (pos >> 6) + h * NB
            return carry

        lax.fori_loop(0, buf_ref.shape[0] // 16, step2, 0)
        pltpu.sync_copy(out_ref, gch_hbm.at[b, h])


def _onehot_chunks_t(g):
    kcol = lax.broadcasted_iota(jnp.int32, (N_HASHES * NB, 1), 0)
    u = jnp.zeros((N_HASHES * NB, g.shape[1]), jnp.float32)
    for h in range(N_HASHES):
        u = u + (g[h:h + 1, :] == kcol).astype(jnp.float32)
    return u.astype(jnp.bfloat16)


def _stage_e(qki_ref, qkn_ref, v_ref, gi_ref, gf_ref,
             wout_ref, bout_ref, out_ref, ut_ref):
    nj = qkn_ref.shape[0] // TBLK

    @pl.when(pl.program_id(1) == 0)
    def _():
        for j in range(nj):
            sl = slice(j * TBLK, (j + 1) * TBLK)
            ut_ref[:, sl] = _onehot_chunks_t(gf_ref[:, sl])

    u_i = _onehot_chunks_t(gi_ref[...])
    qki_bf = qki_ref[...]
    cb = lax.dot_general(u_i, ut_ref[...], (((0,), (0,)), ((), ())),
                         preferred_element_type=jnp.float32)
    sb = lax.dot_general(qki_bf, qkn_ref[...], (((1,), (1,)), ((), ())),
                         preferred_element_type=jnp.float32)
    w = (cb * sb).astype(jnp.bfloat16)
    acc = jnp.dot(w, v_ref[...], preferred_element_type=jnp.float32)
    out_ref[...] = lax.dot_general(
        acc.astype(jnp.bfloat16), wout_ref[...], (((1,), (1,)), ((), ())),
        preferred_element_type=jnp.float32) + bout_ref[...]


def kernel(x, Wqk, bqk, Wv, bv, Wout, bout):
    B, S, D = x.shape
    nblk = S // TBLK
    rot = jax.random.normal(jax.random.key(42), (1, D, N_HASHES, NB // 2),
                            dtype=x.dtype)
    rot2 = rot.reshape(D, N_HASHES * (NB // 2))
    xbf = x.astype(jnp.bfloat16)
    wqkbf = Wqk.astype(jnp.bfloat16)
    wvbf = Wv.astype(jnp.bfloat16)
    woutbf = Wout.astype(jnp.bfloat16)
    rotbf = rot2.astype(jnp.bfloat16)

    qk, qkn, v, bkt = pl.pallas_call(
        _stage_a,
        grid=(B, nblk),
        in_specs=[
            pl.BlockSpec((None, TBLK, D), lambda b, i: (b, i, 0)),
            pl.BlockSpec((D, D), lambda b, i: (0, 0)),
            pl.BlockSpec((1, D), lambda b, i: (0, 0)),
            pl.BlockSpec((D, D), lambda b, i: (0, 0)),
            pl.BlockSpec((1, D), lambda b, i: (0, 0)),
            pl.BlockSpec((D, N_HASHES * (NB // 2)), lambda b, i: (0, 0)),
        ],
        out_specs=[
            pl.BlockSpec((None, TBLK, D), lambda b, i: (b, i, 0)),
            pl.BlockSpec((None, TBLK, D), lambda b, i: (b, i, 0)),
            pl.BlockSpec((None, TBLK, D), lambda b, i: (b, i, 0)),
            pl.BlockSpec((None, N_HASHES, TBLK), lambda b, i: (b, 0, i)),
        ],
        out_shape=[
            jax.ShapeDtypeStruct((B, S, D), jnp.bfloat16),
            jax.ShapeDtypeStruct((B, S, D), jnp.bfloat16),
            jax.ShapeDtypeStruct((B, S, D), jnp.bfloat16),
            jax.ShapeDtypeStruct((B, N_HASHES, S), jnp.int32),
        ],
        compiler_params=pltpu.CompilerParams(
            dimension_semantics=("parallel", "parallel")),
    )(xbf, wqkbf, bqk.reshape(1, D), wvbf, bv.reshape(1, D), rotbf)

    gch = pl.kernel(
        _sc_sort,
        out_type=jax.ShapeDtypeStruct((B, N_HASHES, S), jnp.int32),
        mesh=plsc.VectorSubcoreMesh(core_axis_name="c", subcore_axis_name="s"),
        scratch_types=[
            pltpu.VMEM((S,), jnp.int32),
            pltpu.VMEM((S,), jnp.int32),
            pltpu.VMEM((S,), jnp.int32),
            pltpu.VMEM((NB,), jnp.int32),
            pltpu.VMEM((NB,), jnp.int32),
        ],
        compiler_params=pltpu.CompilerParams(needs_layout_passes=False),
    )(bkt)

    out = pl.pallas_call(
        _stage_e,
        grid=(B, nblk),
        in_specs=[
            pl.BlockSpec((None, TBLK, D), lambda b, i: (b, i, 0)),
            pl.BlockSpec((None, S, D), lambda b, i: (b, 0, 0)),
            pl.BlockSpec((None, S, D), lambda b, i: (b, 0, 0)),
            pl.BlockSpec((None, N_HASHES, TBLK), lambda b, i: (b, 0, i)),
            pl.BlockSpec((None, N_HASHES, S), lambda b, i: (b, 0, 0)),
            pl.BlockSpec((D, D), lambda b, i: (0, 0)),
            pl.BlockSpec((1, D), lambda b, i: (0, 0)),
        ],
        out_specs=pl.BlockSpec((None, TBLK, D), lambda b, i: (b, i, 0)),
        out_shape=jax.ShapeDtypeStruct((B, S, D), jnp.float32),
        scratch_shapes=[pltpu.VMEM((N_HASHES * NB, S), jnp.bfloat16)],
        compiler_params=pltpu.CompilerParams(
            dimension_semantics=("parallel", "arbitrary")),
    )(qk, qkn, v, gch, gch, woutbf, bout.reshape(1, D))

    return out


# speedup vs baseline: 1.1758x; 1.0032x over previous
import functools

import jax
import jax.numpy as jnp
from jax import lax
from jax.experimental import pallas as pl
from jax.experimental.pallas import tpu as pltpu
from jax.experimental.pallas import tpu_sc as plsc

N_HASHES = 8
BUCKET = 64
NB = 32
TBLK = 512

_HIGH = lax.Precision.HIGHEST


def _argmax_first(vals, n):
    m = jnp.max(vals, axis=-1, keepdims=True)
    ids = lax.broadcasted_iota(jnp.int32, vals.shape, vals.ndim - 1)
    cand = jnp.where(vals >= m, ids, n)
    return jnp.min(cand, axis=-1)


def _stage_a(x_ref, wqk_ref, bqk_ref, wv_ref, bv_ref, rot_ref,
             qk_ref, qkn_ref, v_ref, bkt_ref):
    xb = x_ref[...]
    qk = lax.dot_general(xb, wqk_ref[...], (((1,), (1,)), ((), ())),
                         preferred_element_type=jnp.float32) + bqk_ref[...]
    vv = lax.dot_general(xb, wv_ref[...], (((1,), (1,)), ((), ())),
                         preferred_element_type=jnp.float32) + bv_ref[...]
    qkbf = qk.astype(jnp.bfloat16)
    qk_ref[...] = qkbf
    v_ref[...] = vv.astype(jnp.bfloat16)
    nrm = jnp.sqrt(jnp.sum(qk * qk, axis=1, keepdims=True))
    nrm = jnp.maximum(nrm, 1e-12)
    qkn_ref[...] = (qk * ((1.0 / 32.0) / nrm)).astype(jnp.bfloat16)
    rv = jnp.dot(qkbf, rot_ref[...], preferred_element_type=jnp.float32)
    cols = []
    for h in range(N_HASHES):
        sl = rv[:, h * (NB // 2):(h + 1) * (NB // 2)]
        vals = jnp.concatenate([sl, -sl], axis=1)
        cols.append(_argmax_first(vals, NB).reshape(-1, 1))
    colsmat = jnp.concatenate(cols, axis=1).astype(jnp.float32)
    rr = lax.broadcasted_iota(jnp.int32, (TBLK, TBLK), 0)
    cc = lax.broadcasted_iota(jnp.int32, (TBLK, TBLK), 1)
    eye = (rr == cc).astype(jnp.float32)
    bktT = lax.dot_general(colsmat, eye, (((0,), (0,)), ((), ())),
                           preferred_element_type=jnp.float32)
    bkt_ref[...] = bktT.astype(jnp.int32)


def _sc_sort(bkt_hbm, gch_hbm, buf_ref, rank_ref, out_ref, cnt_ref, off_ref):
    wid = lax.axis_index("s") * 2 + lax.axis_index("c")
    b = wid // N_HASHES
    h = wid % N_HASHES
    lid = lax.broadcasted_iota(jnp.int32, (16,), 0)

    @pl.when(wid < 2 * N_HASHES)
    def _():
        pltpu.sync_copy(bkt_hbm.at[b, h], buf_ref)
        cnt_ref[pl.ds(0, 16)] = jnp.zeros((16,), jnp.int32)
        cnt_ref[pl.ds(16, 16)] = jnp.zeros((16,), jnp.int32)

        def step1(t0, carry):
            base = t0 * 16
            bv = buf_ref[pl.ds(base, 16)]
            tot = jnp.zeros((16,), jnp.int32)
            pre = jnp.zeros((16,), jnp.int32)
            for j in range(16):
                bj = plsc.load_gather(buf_ref, [lid * 0 + (base + j)])
                eq = (bv == bj).astype(jnp.int32)
                tot = tot + eq
                pre = pre + jnp.where(lid > j, eq, 0)
            old = plsc.load_gather(cnt_ref, [bv])
            rank_ref[pl.ds(base, 16)] = old + pre
            plsc.store_scatter(cnt_ref, [bv], old + tot,
                               mask=pre == tot - 1)
            return carry

        lax.fori_loop(0, buf_ref.shape[0] // 16, step1, 0)
        c0 = cnt_ref[pl.ds(0, 16)]
        c1 = cnt_ref[pl.ds(16, 16)]
        off_ref[pl.ds(0, 16)] = plsc.cumsum(c0) - c0
        off_ref[pl.ds(16, 16)] = plsc.cumsum(c1) - c1 + jnp.sum(c0)

        def step2(t0, carry):
            base = t0 * 16
            bv = buf_ref[pl.ds(base, 16)]
            pos = plsc.load_gather(off_ref, [bv]) + rank_ref[pl.ds(base, 16)]
            out_ref[pl.ds(base, 16)] = (pos >> 6) + h * NB
            return carry

        lax.fori_loop(0, buf_ref.shape[0] // 16, step2, 0)
        pltpu.sync_copy(out_ref, gch_hbm.at[b, h])


def _onehot_chunks_t(g):
    kcol = lax.broadcasted_iota(jnp.int32, (N_HASHES * NB, 1), 0)
    u = jnp.zeros((N_HASHES * NB, g.shape[1]), jnp.float32)
    for h in range(N_HASHES):
        u = u + (g[h:h + 1, :] == kcol).astype(jnp.float32)
    return u.astype(jnp.bfloat16)


def _stage_e(qki_ref, qkn_ref, v_ref, gi_ref, gf_ref,
             wout_ref, bout_ref, out_ref, ut_ref):
    nj = qkn_ref.shape[0] // TBLK

    @pl.when(pl.program_id(1) == 0)
    def _():
        for j in range(nj):
            sl = slice(j * TBLK, (j + 1) * TBLK)
            ut_ref[:, sl] = _onehot_chunks_t(gf_ref[:, sl])

    u_i = _onehot_chunks_t(gi_ref[...])
    qki_bf = qki_ref[...]
    cb = lax.dot_general(u_i, ut_ref[...], (((0,), (0,)), ((), ())),
                         preferred_element_type=jnp.float32)
    sb = lax.dot_general(qki_bf, qkn_ref[...], (((1,), (1,)), ((), ())),
                         preferred_element_type=jnp.float32)
    w = (cb * sb).astype(jnp.bfloat16)
    acc = jnp.dot(w, v_ref[...], preferred_element_type=jnp.float32)
    out_ref[...] = lax.dot_general(
        acc.astype(jnp.bfloat16), wout_ref[...], (((1,), (1,)), ((), ())),
        preferred_element_type=jnp.float32) + bout_ref[...]


def kernel(x, Wqk, bqk, Wv, bv, Wout, bout):
    B, S, D = x.shape
    nblk = S // TBLK
    rot = jax.random.normal(jax.random.key(42), (1, D, N_HASHES, NB // 2),
                            dtype=x.dtype)
    rot2 = rot.reshape(D, N_HASHES * (NB // 2))
    xbf = x.astype(jnp.bfloat16)
    wqkbf = Wqk.astype(jnp.bfloat16)
    wvbf = Wv.astype(jnp.bfloat16)
    woutbf = Wout.astype(jnp.bfloat16)
    rotbf = rot2.astype(jnp.bfloat16)

    qk, qkn, v, bkt = pl.pallas_call(
        _stage_a,
        grid=(B, nblk),
        in_specs=[
            pl.BlockSpec((None, TBLK, D), lambda b, i: (b, i, 0)),
            pl.BlockSpec((D, D), lambda b, i: (0, 0)),
            pl.BlockSpec((1, D), lambda b, i: (0, 0)),
            pl.BlockSpec((D, D), lambda b, i: (0, 0)),
            pl.BlockSpec((1, D), lambda b, i: (0, 0)),
            pl.BlockSpec((D, N_HASHES * (NB // 2)), lambda b, i: (0, 0)),
        ],
        out_specs=[
            pl.BlockSpec((None, TBLK, D), lambda b, i: (b, i, 0)),
            pl.BlockSpec((None, TBLK, D), lambda b, i: (b, i, 0)),
            pl.BlockSpec((None, TBLK, D), lambda b, i: (b, i, 0)),
            pl.BlockSpec((None, N_HASHES, TBLK), lambda b, i: (b, 0, i)),
        ],
        out_shape=[
            jax.ShapeDtypeStruct((B, S, D), jnp.bfloat16),
            jax.ShapeDtypeStruct((B, S, D), jnp.bfloat16),
            jax.ShapeDtypeStruct((B, S, D), jnp.bfloat16),
            jax.ShapeDtypeStruct((B, N_HASHES, S), jnp.int32),
        ],
        compiler_params=pltpu.CompilerParams(
            dimension_semantics=("parallel", "parallel")),
    )(xbf, wqkbf, bqk.reshape(1, D), wvbf, bv.reshape(1, D), rotbf)

    gch = pl.kernel(
        _sc_sort,
        out_type=jax.ShapeDtypeStruct((B, N_HASHES, S), jnp.int32),
        mesh=plsc.VectorSubcoreMesh(core_axis_name="c", subcore_axis_name="s"),
        scratch_types=[
            pltpu.VMEM((S,), jnp.int32),
            pltpu.VMEM((S,), jnp.int32),
            pltpu.VMEM((S,), jnp.int32),
            pltpu.VMEM((NB,), jnp.int32),
            pltpu.VMEM((NB,), jnp.int32),
        ],
        compiler_params=pltpu.CompilerParams(needs_layout_passes=False),
    )(bkt)

    out = pl.pallas_call(
        _stage_e,
        grid=(B, nblk),
        in_specs=[
            pl.BlockSpec((None, TBLK, D), lambda b, i: (b, i, 0)),
            pl.BlockSpec((None, S, D), lambda b, i: (b, 0, 0)),
            pl.BlockSpec((None, S, D), lambda b, i: (b, 0, 0)),
            pl.BlockSpec((None, N_HASHES, TBLK), lambda b, i: (b, 0, i)),
            pl.BlockSpec((None, N_HASHES, S), lambda b, i: (b, 0, 0)),
            pl.BlockSpec((D, D), lambda b, i: (0, 0)),
            pl.BlockSpec((1, D), lambda b, i: (0, 0)),
        ],
        out_specs=pl.BlockSpec((None, TBLK, D), lambda b, i: (b, i, 0)),
        out_shape=jax.ShapeDtypeStruct((B, S, D), jnp.float32),
        scratch_shapes=[pltpu.VMEM((N_HASHES * NB, S), jnp.bfloat16)],
        compiler_params=pltpu.CompilerParams(
            dimension_semantics=("parallel", "arbitrary")),
    )(qk, qkn, v, gch, gch, woutbf, bout.reshape(1, D))

    return out
